# speedup vs baseline: 63.4703x; 63.4703x over previous
import functools

import jax
import jax.numpy as jnp
from jax.experimental import pallas as pl
from jax.experimental.pallas import tpu as pltpu

K_TOP = 8


def _tc_body(x_ref, out_ref):
    S = x_ref.shape[1]
    C = x_ref.shape[2]
    nstep = S // 8
    neg = jnp.full((8, C), -jnp.inf, dtype=jnp.float32)

    def step(i, T):
        d = x_ref[0, pl.ds(i * 8, 8), :]
        out = []
        for k in range(K_TOP):
            t = T[k]
            out.append(jnp.maximum(t, d))
            if k < K_TOP - 1:
                d = jnp.minimum(t, d)
        return tuple(out)

    T = jax.lax.fori_loop(0, nstep, step, tuple([neg] * K_TOP), unroll=4)

    cand = jnp.concatenate(T, axis=0)
    ridx = jax.lax.broadcasted_iota(jnp.int32, (8 * K_TOP, C), 0)
    outs = []
    for _ in range(K_TOP):
        m = jnp.max(cand, axis=0)
        eq = cand == m[None, :]
        first = jnp.min(jnp.where(eq, ridx, 8 * K_TOP), axis=0)
        cand = jnp.where(eq & (ridx == first[None, :]), -jnp.inf, cand)
        outs.append(m)
    res = jnp.stack(outs, axis=0)
    out_ref[0] = jnp.transpose(res, (1, 0))


def _kmax_tc(x):
    B, S, C = x.shape
    out = pl.pallas_call(
        _tc_body,
        grid=(B,),
        in_specs=[pl.BlockSpec((1, S, C), lambda b: (b, 0, 0))],
        out_specs=pl.BlockSpec((1, C, K_TOP), lambda b: (b, 0, 0)),
        out_shape=jax.ShapeDtypeStruct((B, C, K_TOP), jnp.float32),
    )(x)
    return out.reshape(B, C * K_TOP)


def kernel(inputs):
    return _kmax_tc(inputs)
